# baseline (speedup 1.0000x reference)
import functools

import jax
import jax.numpy as jnp
from jax import lax
from jax.experimental import pallas as pl
from jax.experimental.pallas import tpu as pltpu
from jax.experimental.pallas import tpu_sc as plsc

B = 16384
NW = 32
BPW = B // NW
NU = 1000000


def _sc_gather_one(idx, e0, e1, e2):
    mesh = plsc.VectorSubcoreMesh(core_axis_name="c", subcore_axis_name="s")

    @functools.partial(
        pl.kernel,
        mesh=mesh,
        compiler_params=pltpu.CompilerParams(use_tc_tiling_on_sc=False),
        out_type=jax.ShapeDtypeStruct((3, B), jnp.float32),
        scratch_types=[
            pltpu.VMEM((BPW,), jnp.int32),
            pltpu.VMEM((3, BPW), jnp.float32),
            pltpu.SemaphoreType.DMA,
        ],
    )
    def gather_kernel(idx_hbm, e0_hbm, e1_hbm, e2_hbm, out_hbm,
                      idx_v, row_v, sem):
        wid = lax.axis_index("s") * 2 + lax.axis_index("c")
        base = wid * BPW
        pltpu.sync_copy(idx_hbm.at[pl.ds(base, BPW)], idx_v)
        copies = [
            pltpu.async_copy(e_hbm.at[idx_v], row_v.at[c], sem)
            for c, e_hbm in enumerate((e0_hbm, e1_hbm, e2_hbm))
        ]
        for cp in copies:
            cp.wait()
        for c in range(3):
            pltpu.sync_copy(row_v.at[c], out_hbm.at[c, pl.ds(base, BPW)])

    return gather_kernel(idx, e0, e1, e2)


def _lrelu(v):
    return jnp.where(v >= 0, v, 0.1 * v)


def _mlp_t_body(u_ref, i_ref, a_ref, w1u, w1i, w1a, b1, w2, b2, w3, b3,
                w4, b4, w5, b5, w6, b6, w7, b7, out_ref):
    dot = functools.partial(
        lax.dot_general,
        dimension_numbers=(((0,), (0,)), ((), ())),
        preferred_element_type=jnp.float32,
        precision=lax.Precision.HIGHEST,
    )
    h = (dot(w1u[...], u_ref[...]) + dot(w1i[...], i_ref[...])
         + dot(w1a[...], a_ref[...]) + b1[...])
    h = _lrelu(h)
    h = _lrelu(dot(w2[...], h) + b2[...])
    h = _lrelu(dot(w3[...], h) + b3[...])
    h = dot(w4[...], h) + b4[...]
    h = _lrelu(dot(w5[...], h) + b5[...])
    h = _lrelu(dot(w6[...], h) + b6[...])
    h = dot(w7[...], h) + b7[...]
    out_ref[...] = 5.0 / (1.0 + jnp.exp(-h))


def _mlp_t(u, i, a, *ws, tb=16384):
    def _full(arr):
        return pl.BlockSpec(arr.shape, lambda j: (0,) * arr.ndim)

    in_specs = [
        pl.BlockSpec((3, tb), lambda j: (0, j)),
        pl.BlockSpec((3, tb), lambda j: (0, j)),
        pl.BlockSpec((5, tb), lambda j: (0, j)),
    ] + [_full(w) for w in ws]
    return pl.pallas_call(
        _mlp_t_body,
        grid=(B // tb,),
        in_specs=in_specs,
        out_specs=pl.BlockSpec((1, tb), lambda j: (0, j)),
        out_shape=jax.ShapeDtypeStruct((1, B), jnp.float32),
    )(u, i, a, *ws)


def kernel(x, a, user_emb, item_emb, W1, b1, W2, b2, W3, b3, W4, b4,
           W5, b5, W6, b6, W7, b7):
    user_idx = x[:, 0]
    item_idx = x[:, 1]
    it = item_emb.T
    i = _sc_gather_one(item_idx, it[0], it[1], it[2])
    ut = user_emb.T
    u = _sc_gather_one(user_idx, ut[0], ut[1], ut[2])
    out = _mlp_t(
        u, i, a.T,
        W1[0:3], W1[3:6], W1[6:11], b1.reshape(10, 1),
        W2, b2.reshape(10, 1), W3, b3.reshape(10, 1),
        W4, b4.reshape(10, 1), W5, b5.reshape(10, 1),
        W6, b6.reshape(10, 1), W7, b7.reshape(1, 1),
    )
    return out[0]


# speedup vs baseline: 1.4282x; 1.4282x over previous
import functools

import jax
import jax.numpy as jnp
from jax import lax
from jax.experimental import pallas as pl
from jax.experimental.pallas import tpu as pltpu
from jax.experimental.pallas import tpu_sc as plsc

B = 16384
NW = 32
BPW = B // NW
NU = 1000000


def _sc_gather_one(idx, embf):
    mesh = plsc.VectorSubcoreMesh(core_axis_name="c", subcore_axis_name="s")

    @functools.partial(
        pl.kernel,
        mesh=mesh,
        compiler_params=pltpu.CompilerParams(use_tc_tiling_on_sc=False),
        out_type=jax.ShapeDtypeStruct((3, B), jnp.float32),
        scratch_types=[
            pltpu.VMEM((3, BPW), jnp.int32),
            pltpu.VMEM((3, BPW), jnp.float32),
            pltpu.SemaphoreType.DMA,
        ],
    )
    def gather_kernel(idx_hbm, embf_hbm, out_hbm, idx_v, row_v, sem):
        wid = lax.axis_index("s") * 2 + lax.axis_index("c")
        base = wid * BPW
        pltpu.sync_copy(idx_hbm.at[pl.ds(base, BPW)], idx_v.at[0])
        for g in range(BPW // 16):
            v = idx_v[0, pl.ds(16 * g, 16)]
            idx_v[1, pl.ds(16 * g, 16)] = v + NU
            idx_v[2, pl.ds(16 * g, 16)] = v + 2 * NU
        copies = [
            pltpu.async_copy(embf_hbm.at[idx_v.at[c]], row_v.at[c], sem)
            for c in range(3)
        ]
        for cp in copies:
            cp.wait()
        for c in range(3):
            pltpu.sync_copy(row_v.at[c], out_hbm.at[c, pl.ds(base, BPW)])

    return gather_kernel(idx, embf)


def _lrelu(v):
    return jnp.where(v >= 0, v, 0.1 * v)


def _mlp_t_body(u_ref, i_ref, a_ref, w1u, w1i, w1a, w2, w3,
                w4, w5, w6, w7, ball, out_ref):
    dot = functools.partial(
        lax.dot_general,
        dimension_numbers=(((0,), (0,)), ((), ())),
        preferred_element_type=jnp.float32,
    )
    bs = ball[...]
    h = (dot(w1u[...], u_ref[...]) + dot(w1i[...], i_ref[...])
         + dot(w1a[...], a_ref[...]) + bs[:, 0:1])
    h = _lrelu(h)
    h = _lrelu(dot(w2[...], h) + bs[:, 1:2])
    h = _lrelu(dot(w3[...], h) + bs[:, 2:3])
    h = dot(w4[...], h) + bs[:, 3:4]
    h = _lrelu(dot(w5[...], h) + bs[:, 4:5])
    h = _lrelu(dot(w6[...], h) + bs[:, 5:6])
    h = dot(w7[...], h) + bs[0:1, 6:7]
    out_ref[...] = 5.0 / (1.0 + jnp.exp(-h))


def _mlp_t(u, i, a, *ws, tb=16384):
    def _full(arr):
        return pl.BlockSpec(arr.shape, lambda j: (0,) * arr.ndim)

    in_specs = [
        pl.BlockSpec((3, tb), lambda j: (0, j)),
        pl.BlockSpec((3, tb), lambda j: (0, j)),
        pl.BlockSpec((5, tb), lambda j: (0, j)),
    ] + [_full(w) for w in ws]
    return pl.pallas_call(
        _mlp_t_body,
        grid=(B // tb,),
        in_specs=in_specs,
        out_specs=pl.BlockSpec((1, tb), lambda j: (0, j)),
        out_shape=jax.ShapeDtypeStruct((1, B), jnp.float32),
    )(u, i, a, *ws)


def kernel(x, a, user_emb, item_emb, W1, b1, W2, b2, W3, b3, W4, b4,
           W5, b5, W6, b6, W7, b7):
    user_idx = x[:, 0]
    item_idx = x[:, 1]
    iembf = item_emb.T.reshape(-1)
    i = _sc_gather_one(item_idx, iembf)
    uembf = user_emb.T.reshape(-1)
    u = _sc_gather_one(user_idx, uembf)
    ball = jnp.stack(
        [b1, b2, b3, b4, b5, b6,
         jnp.broadcast_to(b7, (10,))], axis=1)
    out = _mlp_t(
        u, i, a.T,
        W1[0:3], W1[3:6], W1[6:11], W2, W3, W4, W5, W6, W7, ball,
    )
    return out[0]
